# baseline (speedup 1.0000x reference)
import jax
import jax.numpy as jnp
from jax import lax
from jax.experimental import pallas as pl
from jax.experimental.pallas import tpu as pltpu
from jax.experimental.pallas import tpu_sc as plsc

_NUM_USERS = 1000000
_NUM_ITEMS = 1000000
_EMBED_DIM = 32
_TRAIN_INTERACTION_SIZE = 100000000
_WEIGHT_DECAY = 1e-4
_BATCH = 16384

_NC = 2
_NS = 16
_NW = _NC * _NS
_BPW = _BATCH // _NW

_SPARSITY = _TRAIN_INTERACTION_SIZE / (_NUM_USERS * _NUM_ITEMS)
_BPR_SCALE = 1.0 / (_NUM_USERS * _SPARSITY * _BATCH)
_REG_SCALE = _WEIGHT_DECAY * 0.5 / _BATCH


def _sc_body(users_hbm, pos_hbm, neg_hbm, uemb_hbm, iemb_hbm, pop_hbm,
             urows_out, prows_out, nrows_out, alpha_out,
             uidx_v, pidx_v, nidx_v, urows_v, prows_v, nrows_v, alpha_v,
             sem_u, sem_p, sem_n, sem_a):
    wid = lax.axis_index("s") * _NC + lax.axis_index("c")
    base = wid * _BPW

    pltpu.sync_copy(users_hbm.at[pl.ds(base, _BPW)], uidx_v)
    pltpu.sync_copy(pos_hbm.at[pl.ds(base, _BPW)], pidx_v)
    pltpu.sync_copy(neg_hbm.at[pl.ds(base, _BPW)], nidx_v)

    cu = pltpu.async_copy(uemb_hbm.at[uidx_v], urows_v, sem_u)
    cp = pltpu.async_copy(iemb_hbm.at[pidx_v], prows_v, sem_p)
    cn = pltpu.async_copy(iemb_hbm.at[nidx_v], nrows_v, sem_n)
    ca = pltpu.async_copy(pop_hbm.at[nidx_v], alpha_v, sem_a)
    cu.wait()
    cp.wait()
    cn.wait()
    ca.wait()

    pltpu.sync_copy(urows_v, urows_out.at[pl.ds(base, _BPW)])
    pltpu.sync_copy(prows_v, prows_out.at[pl.ds(base, _BPW)])
    pltpu.sync_copy(nrows_v, nrows_out.at[pl.ds(base, _BPW)])
    pltpu.sync_copy(alpha_v, alpha_out.at[pl.ds(base, _BPW)])


_sc_gather = pl.kernel(
    _sc_body,
    out_type=[
        jax.ShapeDtypeStruct((_BATCH, _EMBED_DIM), jnp.float32),
        jax.ShapeDtypeStruct((_BATCH, _EMBED_DIM), jnp.float32),
        jax.ShapeDtypeStruct((_BATCH, _EMBED_DIM), jnp.float32),
        jax.ShapeDtypeStruct((_BATCH,), jnp.float32),
    ],
    mesh=plsc.VectorSubcoreMesh(core_axis_name="c", subcore_axis_name="s"),
    compiler_params=pltpu.CompilerParams(use_tc_tiling_on_sc=False),
    scratch_types=[
        pltpu.VMEM((_BPW,), jnp.int32),
        pltpu.VMEM((_BPW,), jnp.int32),
        pltpu.VMEM((_BPW,), jnp.int32),
        pltpu.VMEM((_BPW, _EMBED_DIM), jnp.float32),
        pltpu.VMEM((_BPW, _EMBED_DIM), jnp.float32),
        pltpu.VMEM((_BPW, _EMBED_DIM), jnp.float32),
        pltpu.VMEM((_BPW,), jnp.float32),
        pltpu.SemaphoreType.DMA,
        pltpu.SemaphoreType.DMA,
        pltpu.SemaphoreType.DMA,
        pltpu.SemaphoreType.DMA,
    ],
)


def _loss_body(urows_ref, prows_ref, nrows_ref, alpha_ref, out_ref):
    u = urows_ref[...]
    p = prows_ref[...]
    q = nrows_ref[...]
    a = alpha_ref[...]
    x = jnp.sum(u * (q - p), axis=1)
    sp = jnp.maximum(x, 0.0) + jnp.log(1.0 + jnp.exp(-jnp.abs(x)))
    wb = jnp.sum(a.reshape(-1) * sp)
    ssq = jnp.sum(u * u) + jnp.sum(p * p) + jnp.sum(q * q)
    out_ref[0, 0] = wb * _BPR_SCALE + ssq * _REG_SCALE


_loss_call = pl.pallas_call(
    _loss_body,
    out_shape=jax.ShapeDtypeStruct((1, 1), jnp.float32),
    in_specs=[
        pl.BlockSpec(memory_space=pltpu.VMEM),
        pl.BlockSpec(memory_space=pltpu.VMEM),
        pl.BlockSpec(memory_space=pltpu.VMEM),
        pl.BlockSpec(memory_space=pltpu.VMEM),
    ],
    out_specs=pl.BlockSpec(memory_space=pltpu.SMEM),
)


def kernel(users, positive_items, negative_items, user_embedding,
           item_embedding, popularity):
    urows, prows, nrows, alpha = _sc_gather(
        users, positive_items, negative_items,
        user_embedding, item_embedding, popularity)
    loss = _loss_call(urows, prows, nrows, alpha.reshape(128, 128))
    return loss[0, 0]


# speedup vs baseline: 1.6238x; 1.6238x over previous
import jax
import jax.numpy as jnp
from jax import lax
from jax.experimental import pallas as pl
from jax.experimental.pallas import tpu as pltpu
from jax.experimental.pallas import tpu_sc as plsc

_NUM_USERS = 1000000
_NUM_ITEMS = 1000000
_EMBED_DIM = 32
_TRAIN_INTERACTION_SIZE = 100000000
_WEIGHT_DECAY = 1e-4
_BATCH = 16384

_NC = 2
_NS = 16
_NW = _NC * _NS
_BPW = _BATCH // _NW
_RPW = _BPW * _EMBED_DIM // 128

_SPARSITY = _TRAIN_INTERACTION_SIZE / (_NUM_USERS * _NUM_ITEMS)
_BPR_SCALE = 1.0 / (_NUM_USERS * _SPARSITY * _BATCH)
_REG_SCALE = _WEIGHT_DECAY * 0.5 / _BATCH


def _sc_body(users_hbm, pos_hbm, neg_hbm, uemb_hbm, iemb_hbm, pop_hbm,
             urows_out, prows_out, nrows_out, alpha_out,
             uidx_v, pidx_v, nidx_v, urows_v, prows_v, nrows_v, alpha_v,
             sem_u, sem_p, sem_n, sem_a):
    wid = lax.axis_index("s") * _NC + lax.axis_index("c")
    base = wid * _BPW

    pltpu.sync_copy(users_hbm.at[pl.ds(base, _BPW)], uidx_v)
    pltpu.sync_copy(pos_hbm.at[pl.ds(base, _BPW)], pidx_v)
    pltpu.sync_copy(neg_hbm.at[pl.ds(base, _BPW)], nidx_v)

    def issue(g, carry):
        iu_vec = uidx_v[pl.ds(g * 16, 16)]
        ip_vec = pidx_v[pl.ds(g * 16, 16)]
        iq_vec = nidx_v[pl.ds(g * 16, 16)]
        pltpu.async_copy(pop_hbm.at[iq_vec],
                         alpha_v.at[g // 8, pl.ds((g % 8) * 16, 16)], sem_a)
        for k in range(16):
            b = g * 16 + k
            iu = iu_vec[k]
            ip = ip_vec[k]
            iq = iq_vec[k]
            r = b // 4
            c = (b % 4) * _EMBED_DIM
            pltpu.async_copy(uemb_hbm.at[iu],
                             urows_v.at[r, pl.ds(c, _EMBED_DIM)], sem_u)
            pltpu.async_copy(iemb_hbm.at[ip],
                             prows_v.at[r, pl.ds(c, _EMBED_DIM)], sem_p)
            pltpu.async_copy(iemb_hbm.at[iq],
                             nrows_v.at[r, pl.ds(c, _EMBED_DIM)], sem_n)
        return carry

    lax.fori_loop(0, _BPW // 16, issue, 0)

    pltpu.make_async_copy(urows_out.at[pl.ds(0, _RPW), :], urows_v,
                          sem_u).wait()
    pltpu.make_async_copy(prows_out.at[pl.ds(0, _RPW), :], prows_v,
                          sem_p).wait()
    pltpu.make_async_copy(nrows_out.at[pl.ds(0, _RPW), :], nrows_v,
                          sem_n).wait()
    pltpu.make_async_copy(urows_out.at[pl.ds(0, _BPW // 128), :], alpha_v,
                          sem_a).wait()

    out_base = wid * _RPW
    pltpu.sync_copy(urows_v, urows_out.at[pl.ds(out_base, _RPW)])
    pltpu.sync_copy(prows_v, prows_out.at[pl.ds(out_base, _RPW)])
    pltpu.sync_copy(nrows_v, nrows_out.at[pl.ds(out_base, _RPW)])
    pltpu.sync_copy(alpha_v, alpha_out.at[pl.ds(wid * (_BPW // 128),
                                                _BPW // 128)])


_sc_gather = pl.kernel(
    _sc_body,
    out_type=[
        jax.ShapeDtypeStruct((_NW * _RPW, 128), jnp.float32),
        jax.ShapeDtypeStruct((_NW * _RPW, 128), jnp.float32),
        jax.ShapeDtypeStruct((_NW * _RPW, 128), jnp.float32),
        jax.ShapeDtypeStruct((_BATCH // 128, 128), jnp.float32),
    ],
    mesh=plsc.VectorSubcoreMesh(core_axis_name="c", subcore_axis_name="s"),
    scratch_types=[
        pltpu.VMEM((_BPW,), jnp.int32),
        pltpu.VMEM((_BPW,), jnp.int32),
        pltpu.VMEM((_BPW,), jnp.int32),
        pltpu.VMEM((_RPW, 128), jnp.float32),
        pltpu.VMEM((_RPW, 128), jnp.float32),
        pltpu.VMEM((_RPW, 128), jnp.float32),
        pltpu.VMEM((_BPW // 128, 128), jnp.float32),
        pltpu.SemaphoreType.DMA,
        pltpu.SemaphoreType.DMA,
        pltpu.SemaphoreType.DMA,
        pltpu.SemaphoreType.DMA,
    ],
)


def _loss_body(u_ref, p_ref, n_ref, a_ref, out_ref):
    u = u_ref[...]
    p = p_ref[...]
    q = n_ref[...]
    x = u * (q - p)
    rows = lax.broadcasted_iota(jnp.int32, (128, 4), 0) // _EMBED_DIM
    cols = lax.broadcasted_iota(jnp.int32, (128, 4), 1)
    seg = (rows == cols).astype(jnp.float32)
    d4 = jnp.dot(x, seg, preferred_element_type=jnp.float32)
    sp = jnp.maximum(d4, 0.0) + jnp.log(1.0 + jnp.exp(-jnp.abs(d4)))
    wb = jnp.sum(a_ref[...] * sp)
    ssq = jnp.sum(u * u) + jnp.sum(p * p) + jnp.sum(q * q)
    out_ref[0, 0] = wb * _BPR_SCALE + ssq * _REG_SCALE


_loss_call = pl.pallas_call(
    _loss_body,
    out_shape=jax.ShapeDtypeStruct((1, 1), jnp.float32),
    in_specs=[
        pl.BlockSpec(memory_space=pltpu.VMEM),
        pl.BlockSpec(memory_space=pltpu.VMEM),
        pl.BlockSpec(memory_space=pltpu.VMEM),
        pl.BlockSpec(memory_space=pltpu.VMEM),
    ],
    out_specs=pl.BlockSpec(memory_space=pltpu.SMEM),
)


def kernel(users, positive_items, negative_items, user_embedding,
           item_embedding, popularity):
    urows, prows, nrows, alpha = _sc_gather(
        users, positive_items, negative_items,
        user_embedding, item_embedding, popularity)
    alpha4 = alpha.reshape(_BATCH).reshape(_BATCH // 4, 4)
    loss = _loss_call(urows, prows, nrows, alpha4)
    return loss[0, 0]
